# baseline (speedup 1.0000x reference)
import jax
import jax.numpy as jnp
from jax.experimental import pallas as pl
from jax.experimental.pallas import tpu as pltpu

DIM = 4096
NUM_EXPERTS = 64
BLOCK_T = 1024
NBUF = 3
T_TOKENS = 32768
NB = T_TOKENS // BLOCK_T


def _router_body(x_hbm, w_ref, topi_ref, wts_ref, pri_ref, xbuf, sems):
    def copy(i):
        return pltpu.make_async_copy(
            x_hbm.at[pl.ds(i * BLOCK_T, BLOCK_T), :],
            xbuf.at[i % NBUF],
            sems.at[i % NBUF],
        )

    for i in range(NBUF):
        copy(i).start()

    for i in range(NB):
        copy(i).wait()
        logits = jax.lax.dot_general(
            xbuf[i % NBUF], w_ref[...],
            dimension_numbers=(((1,), (1,)), ((), ())),
            preferred_element_type=jnp.float32,
        )
        if i + NBUF < NB:
            copy(i + NBUF).start()
        m = jnp.max(logits, axis=1, keepdims=True)
        idx = jnp.argmax(logits, axis=1)
        s = jnp.sum(jnp.exp(logits - m), axis=1)
        topi_ref[i:i + 1, :] = idx.astype(jnp.int32).reshape(1, BLOCK_T)
        wts_ref[i:i + 1, :] = (1.0 / s).reshape(1, BLOCK_T)
        rows = i * BLOCK_T + jax.lax.broadcasted_iota(
            jnp.int32, (1, BLOCK_T), 1)
        pri_ref[i:i + 1, :] = -rows.astype(jnp.float32)


@jax.jit
def kernel(x, gate_w):
    t = x.shape[0]
    topi, wts, pri = pl.pallas_call(
        _router_body,
        in_specs=[
            pl.BlockSpec(memory_space=pl.ANY),
            pl.BlockSpec(memory_space=pltpu.VMEM),
        ],
        out_specs=[
            pl.BlockSpec(memory_space=pltpu.VMEM),
            pl.BlockSpec(memory_space=pltpu.VMEM),
            pl.BlockSpec(memory_space=pltpu.VMEM),
        ],
        out_shape=[
            jax.ShapeDtypeStruct((NB, BLOCK_T), jnp.int32),
            jax.ShapeDtypeStruct((NB, BLOCK_T), jnp.float32),
            jax.ShapeDtypeStruct((NB, BLOCK_T), jnp.float32),
        ],
        scratch_shapes=[
            pltpu.VMEM((NBUF, BLOCK_T, DIM), jnp.float32),
            pltpu.SemaphoreType.DMA((NBUF,)),
        ],
        compiler_params=pltpu.CompilerParams(
            vmem_limit_bytes=128 * 1024 * 1024),
    )(x, gate_w)
    return (topi.reshape(t, 1), wts.reshape(t, 1), pri.reshape(t))


# speedup vs baseline: 1.3006x; 1.3006x over previous
import functools

import jax
import jax.numpy as jnp
from jax.experimental import pallas as pl
from jax.experimental.pallas import tpu as pltpu

DIM = 4096
NUM_EXPERTS = 64
BLOCK_T = 1024
KHALF = DIM // 2


def _router_body(x0_ref, x1_ref, w_ref, topi_ref, wts_ref, pri_ref, *, block_t):
    dn = (((1,), (1,)), ((), ()))
    logits = jax.lax.dot_general(
        x0_ref[...], w_ref[:, :KHALF], dn, preferred_element_type=jnp.float32)
    logits += jax.lax.dot_general(
        x1_ref[...], w_ref[:, KHALF:], dn, preferred_element_type=jnp.float32)
    m = jnp.max(logits, axis=1, keepdims=True)
    idx = jnp.argmax(logits, axis=1)
    s = jnp.sum(jnp.exp(logits - m), axis=1)
    topi_ref[...] = idx.astype(jnp.int32).reshape(1, 1, block_t)
    wts_ref[...] = (1.0 / s).reshape(1, 1, block_t)
    row0 = pl.program_id(0) * block_t
    rows = row0 + jax.lax.broadcasted_iota(jnp.int32, (1, 1, block_t), 2)
    pri_ref[...] = -rows.astype(jnp.float32)


@jax.jit
def kernel(x, gate_w):
    t = x.shape[0]
    nb = t // BLOCK_T
    grid = (nb,)
    topi, wts, pri = pl.pallas_call(
        functools.partial(_router_body, block_t=BLOCK_T),
        grid=grid,
        in_specs=[
            pl.BlockSpec((BLOCK_T, KHALF), lambda i: (i, 0)),
            pl.BlockSpec((BLOCK_T, KHALF), lambda i: (i, 1)),
            pl.BlockSpec((NUM_EXPERTS, DIM), lambda i: (0, 0)),
        ],
        out_specs=[
            pl.BlockSpec((1, 1, BLOCK_T), lambda i: (i, 0, 0)),
            pl.BlockSpec((1, 1, BLOCK_T), lambda i: (i, 0, 0)),
            pl.BlockSpec((1, 1, BLOCK_T), lambda i: (i, 0, 0)),
        ],
        out_shape=[
            jax.ShapeDtypeStruct((nb, 1, BLOCK_T), jnp.int32),
            jax.ShapeDtypeStruct((nb, 1, BLOCK_T), jnp.float32),
            jax.ShapeDtypeStruct((nb, 1, BLOCK_T), jnp.float32),
        ],
        compiler_params=pltpu.CompilerParams(
            vmem_limit_bytes=128 * 1024 * 1024),
    )(x, x, gate_w)
    return (topi.reshape(t, 1), wts.reshape(t, 1), pri.reshape(t))
